# Initial kernel scaffold; baseline (speedup 1.0000x reference)
import jax
import jax.numpy as jnp
from jax.experimental import pallas as pl


def kernel(x, edge_index, batch):
    raise NotImplementedError("write your pallas kernel here")


# speedup vs baseline: 2.8074x; 2.8074x over previous
import functools

import jax
import jax.numpy as jnp
from jax import lax
from jax.experimental import pallas as pl
from jax.experimental.pallas import tpu as pltpu
from jax.experimental.pallas import tpu_sc as plsc

_NN = 100000
_F = 128
_G = 512
_NC = 2
_NS = 16
_L = 16
_FH = _F // _NC
_CHUNK = 128
_NCH = (_NN + _CHUNK - 1) // _CHUNK
_FULL_STEPS = (_NCH - 1) // _NS
_LAST_FULL = (_NCH - 1) - _FULL_STEPS * _NS
_TAIL = _NN - (_NCH - 1) * _CHUNK
_DUMMY = _G
_ACC_ROWS = _G + _NS
_ZROWS = _ACC_ROWS // _NS
_OROWS = _G // _NS
_STEPS_PER_TILE = _FULL_STEPS + 1

_mesh = plsc.VectorSubcoreMesh(core_axis_name="c", subcore_axis_name="s")


@functools.partial(
    pl.kernel,
    out_type=jax.ShapeDtypeStruct((_G, _F), jnp.float32),
    mesh=_mesh,
    scratch_types=[
        pltpu.VMEM((_STEPS_PER_TILE, _CHUNK), jnp.int32),
        pltpu.VMEM((_CHUNK, _FH), jnp.float32),
        pltpu.VMEM_SHARED((_ACC_ROWS, _FH), jnp.float32),
    ],
    compiler_params=pltpu.CompilerParams(use_tc_tiling_on_sc=False),
)
def _segsum(x_hbm, b_hbm, out_hbm, idx_v, rows_v, acc_sh):
    cid = lax.axis_index("c")
    sid = lax.axis_index("s")
    col0 = cid * _FH

    zero = jnp.zeros((_L,), jnp.float32)
    for i in range(_ZROWS):
        for j in range(_FH // _L):
            rows_v[i, pl.ds(j * _L, _L)] = zero
    pltpu.sync_copy(
        rows_v.at[pl.ds(0, _ZROWS)],
        acc_sh.at[pl.ds(sid * _ZROWS, _ZROWS)],
    )
    dummy = jnp.full((_L,), _DUMMY, jnp.int32)
    for j in range(_CHUNK // _L):
        idx_v[_FULL_STEPS, pl.ds(j * _L, _L)] = dummy
    plsc.subcore_barrier()

    def step(g, carry):
        base = (g * _NS + sid) * _CHUNK
        pltpu.sync_copy(b_hbm.at[pl.ds(base, _CHUNK)], idx_v.at[g])
        pltpu.sync_copy(
            x_hbm.at[pl.ds(base, _CHUNK), pl.ds(col0, _FH)], rows_v
        )
        pltpu.sync_copy(rows_v, acc_sh.at[idx_v.at[g]], add=True)
        return carry

    lax.fori_loop(0, _FULL_STEPS, step, 0)

    @pl.when(sid < _LAST_FULL)
    def _():
        step(_FULL_STEPS, 0)

    @pl.when(sid == _LAST_FULL)
    def _():
        base = (_NCH - 1) * _CHUNK
        pltpu.sync_copy(
            b_hbm.at[pl.ds(base, _TAIL)],
            idx_v.at[_FULL_STEPS, pl.ds(0, _TAIL)],
        )
        pltpu.sync_copy(
            x_hbm.at[pl.ds(base, _TAIL), pl.ds(col0, _FH)],
            rows_v.at[pl.ds(0, _TAIL)],
        )
        pltpu.sync_copy(rows_v, acc_sh.at[idx_v.at[_FULL_STEPS]], add=True)

    plsc.subcore_barrier()
    pltpu.sync_copy(
        acc_sh.at[pl.ds(sid * _OROWS, _OROWS)],
        out_hbm.at[pl.ds(sid * _OROWS, _OROWS), pl.ds(col0, _FH)],
    )


def kernel(x, edge_index, batch):
    m = _segsum(x, batch)
    return (m, x)


# speedup vs baseline: 3.6056x; 1.2843x over previous
import functools

import jax
import jax.numpy as jnp
from jax import lax
from jax.experimental import pallas as pl
from jax.experimental.pallas import tpu as pltpu
from jax.experimental.pallas import tpu_sc as plsc

_NN = 100000
_F = 128
_G = 512
_NC = 2
_NS = 16
_L = 16
_FH = _F // _NC
_CHUNK = 128
_SUPC = 4
_SROWS = _SUPC * _CHUNK
_NBUF = 3
_NSUP = (_NN + _SROWS - 1) // _SROWS
_TFULL = (_NSUP - 1) // _NS
_REM = (_NSUP - 1) - _TFULL * _NS
_TROWS = _NN - (_NSUP - 1) * _SROWS
_TAIL = _NN - (_NN // _CHUNK) * _CHUNK
_TAIL_CH = (_TROWS + _CHUNK - 1) // _CHUNK
_DUMMY = _G
_ACC_ROWS = _G + _NS
_ZROWS = _ACC_ROWS // _NS
_OROWS = _G // _NS

_mesh = plsc.VectorSubcoreMesh(core_axis_name="c", subcore_axis_name="s")


@functools.partial(
    pl.kernel,
    out_type=jax.ShapeDtypeStruct((_G, _F), jnp.float32),
    mesh=_mesh,
    scratch_types=[
        pltpu.VMEM((_NBUF * _SUPC, _CHUNK), jnp.int32),
        pltpu.VMEM((_NBUF, _SROWS, _FH), jnp.float32),
        pltpu.VMEM_SHARED((_ACC_ROWS, _FH), jnp.float32),
    ]
    + [pltpu.SemaphoreType.DMA] * (2 * _NBUF),
    compiler_params=pltpu.CompilerParams(use_tc_tiling_on_sc=False),
)
def _segsum(x_hbm, b_hbm, out_hbm, idx_v, rows_v, acc_sh, *sems):
    load_sems, add_sems = sems[:_NBUF], sems[_NBUF:]
    cid = lax.axis_index("c")
    sid = lax.axis_index("s")
    col0 = cid * _FH

    zero = jnp.zeros((_L,), jnp.float32)
    for i in range(_ZROWS):
        for j in range(_FH // _L):
            rows_v[0, i, pl.ds(j * _L, _L)] = zero
    pltpu.sync_copy(
        rows_v.at[0, pl.ds(0, _ZROWS)],
        acc_sh.at[pl.ds(sid * _ZROWS, _ZROWS)],
    )

    def issue_load(t, b):
        base = (t * _NS + sid) * _SROWS
        for j in range(_SUPC):
            pltpu.sync_copy(
                b_hbm.at[pl.ds(base + j * _CHUNK, _CHUNK)],
                idx_v.at[b * _SUPC + j],
            )
        return pltpu.async_copy(
            x_hbm.at[pl.ds(base, _SROWS), pl.ds(col0, _FH)],
            rows_v.at[b],
            load_sems[b],
        )

    def issue_scatter(b):
        return [
            pltpu.async_copy(
                rows_v.at[b, pl.ds(j * _CHUNK, _CHUNK)],
                acc_sh.at[idx_v.at[b * _SUPC + j]],
                add_sems[b],
                add=True,
            )
            for j in range(_SUPC)
        ]

    loads = {t: issue_load(t, t) for t in range(_NBUF)}
    plsc.subcore_barrier()

    for t in range(_TFULL):
        b = t % _NBUF
        loads[t].wait()
        for d in issue_scatter(b):
            d.wait()
        if t + _NBUF < _TFULL:
            loads[t + _NBUF] = issue_load(t + _NBUF, b)

    @pl.when(sid < _REM)
    def _():
        base = (_TFULL * _NS + sid) * _SROWS
        for j in range(_SUPC):
            pltpu.sync_copy(
                b_hbm.at[pl.ds(base + j * _CHUNK, _CHUNK)], idx_v.at[j]
            )
        pltpu.sync_copy(
            x_hbm.at[pl.ds(base, _SROWS), pl.ds(col0, _FH)], rows_v.at[0]
        )
        for j in range(_SUPC):
            pltpu.sync_copy(
                rows_v.at[0, pl.ds(j * _CHUNK, _CHUNK)],
                acc_sh.at[idx_v.at[j]],
                add=True,
            )

    @pl.when(sid == _REM)
    def _():
        base = (_NSUP - 1) * _SROWS
        dummy = jnp.full((_L,), _DUMMY, jnp.int32)
        pltpu.sync_copy(b_hbm.at[pl.ds(base, _CHUNK)], idx_v.at[0])
        for jj in range(_CHUNK // _L):
            idx_v[1, pl.ds(jj * _L, _L)] = dummy
        pltpu.sync_copy(
            b_hbm.at[pl.ds(base + _CHUNK, _TAIL)],
            idx_v.at[1, pl.ds(0, _TAIL)],
        )
        pltpu.sync_copy(
            x_hbm.at[pl.ds(base, _TROWS), pl.ds(col0, _FH)],
            rows_v.at[0, pl.ds(0, _TROWS)],
        )
        for j in range(_TAIL_CH):
            pltpu.sync_copy(
                rows_v.at[0, pl.ds(j * _CHUNK, _CHUNK)],
                acc_sh.at[idx_v.at[j]],
                add=True,
            )

    plsc.subcore_barrier()
    pltpu.sync_copy(
        acc_sh.at[pl.ds(sid * _OROWS, _OROWS)],
        out_hbm.at[pl.ds(sid * _OROWS, _OROWS), pl.ds(col0, _FH)],
    )


def kernel(x, edge_index, batch):
    m = _segsum(x, batch)
    return (m, x)


# speedup vs baseline: 3.6651x; 1.0165x over previous
import functools

import jax
import jax.numpy as jnp
from jax import lax
from jax.experimental import pallas as pl
from jax.experimental.pallas import tpu as pltpu
from jax.experimental.pallas import tpu_sc as plsc

_NN = 100000
_F = 128
_G = 512
_NC = 2
_NS = 16
_L = 16
_FH = _F // _NC
_CHUNK = 128
_NCH = (_NN + _CHUNK - 1) // _CHUNK
_CPT = (_NCH + _NS - 1) // _NS
_PIPE = 46
_NBUF = 3
_TAIL = _NN - (_NCH - 1) * _CHUNK
_DUMMY = _G
_ACC_ROWS = _G + _NS
_ZROWS = _ACC_ROWS // _NS
_OROWS = _G // _NS

_mesh = plsc.VectorSubcoreMesh(core_axis_name="c", subcore_axis_name="s")


@functools.partial(
    pl.kernel,
    out_type=jax.ShapeDtypeStruct((_G, _F), jnp.float32),
    mesh=_mesh,
    scratch_types=[
        pltpu.VMEM((_NBUF, _CHUNK), jnp.int32),
        pltpu.VMEM((_NBUF, _CHUNK, _FH), jnp.float32),
        pltpu.VMEM_SHARED((_ACC_ROWS, _FH), jnp.float32),
    ]
    + [pltpu.SemaphoreType.DMA] * (2 * _NBUF),
    compiler_params=pltpu.CompilerParams(use_tc_tiling_on_sc=False),
)
def _segsum(x_hbm, b_hbm, out_hbm, idx_v, rows_v, acc_sh, *sems):
    load_sems, add_sems = sems[:_NBUF], sems[_NBUF:]
    cid = lax.axis_index("c")
    sid = lax.axis_index("s")
    col0 = cid * _FH

    zero = jnp.zeros((_L,), jnp.float32)
    for i in range(_ZROWS):
        for j in range(_FH // _L):
            rows_v[0, i, pl.ds(j * _L, _L)] = zero
    pltpu.sync_copy(
        rows_v.at[0, pl.ds(0, _ZROWS)],
        acc_sh.at[pl.ds(sid * _ZROWS, _ZROWS)],
    )

    def issue_load(g, b):
        base = (sid * _CPT + g) * _CHUNK
        pltpu.sync_copy(b_hbm.at[pl.ds(base, _CHUNK)], idx_v.at[b])
        return pltpu.async_copy(
            x_hbm.at[pl.ds(base, _CHUNK), pl.ds(col0, _FH)],
            rows_v.at[b],
            load_sems[b],
        )

    loads = {g: issue_load(g, g) for g in range(_NBUF)}
    plsc.subcore_barrier()

    for g in range(_PIPE):
        b = g % _NBUF
        with jax.named_scope("ld_wait"):
            loads[g].wait()
        d = pltpu.async_copy(
            rows_v.at[b], acc_sh.at[idx_v.at[b]], add_sems[b], add=True
        )
        with jax.named_scope("sc_wait"):
            d.wait()
        if g + _NBUF < _PIPE:
            loads[g + _NBUF] = issue_load(g + _NBUF, b)

    with jax.named_scope("tail"):

        @pl.when(sid < _NS - 1)
        def _():
            for g in range(_PIPE, _CPT):
                base = (sid * _CPT + g) * _CHUNK
                pltpu.sync_copy(b_hbm.at[pl.ds(base, _CHUNK)], idx_v.at[0])
                pltpu.sync_copy(
                    x_hbm.at[pl.ds(base, _CHUNK), pl.ds(col0, _FH)],
                    rows_v.at[0],
                )
                pltpu.sync_copy(
                    rows_v.at[0], acc_sh.at[idx_v.at[0]], add=True
                )

        @pl.when(sid == _NS - 1)
        def _():
            base = (_NCH - 1) * _CHUNK
            dummy = jnp.full((_L,), _DUMMY, jnp.int32)
            for jj in range(_CHUNK // _L):
                idx_v[0, pl.ds(jj * _L, _L)] = dummy
            pltpu.sync_copy(
                b_hbm.at[pl.ds(base, _TAIL)], idx_v.at[0, pl.ds(0, _TAIL)]
            )
            pltpu.sync_copy(
                x_hbm.at[pl.ds(base, _TAIL), pl.ds(col0, _FH)],
                rows_v.at[0, pl.ds(0, _TAIL)],
            )
            pltpu.sync_copy(rows_v.at[0], acc_sh.at[idx_v.at[0]], add=True)

    plsc.subcore_barrier()
    pltpu.sync_copy(
        acc_sh.at[pl.ds(sid * _OROWS, _OROWS)],
        out_hbm.at[pl.ds(sid * _OROWS, _OROWS), pl.ds(col0, _FH)],
    )


def kernel(x, edge_index, batch):
    m = _segsum(x, batch)
    return (m, x)


# speedup vs baseline: 4.0049x; 1.0927x over previous
import functools

import jax
import jax.numpy as jnp
from jax import lax
from jax.experimental import pallas as pl
from jax.experimental.pallas import tpu as pltpu
from jax.experimental.pallas import tpu_sc as plsc

_NN = 100000
_F = 128
_G = 512
_NC = 2
_NS = 16
_L = 16
_FH = _F // _NC
_CHUNK = 128
_NCH = (_NN + _CHUNK - 1) // _CHUNK
_CPT = (_NCH + _NS - 1) // _NS
_NCH_PAD = _CPT * _NS
_TAIL = _NN - (_NCH - 1) * _CHUNK
_DUMMY = _G
_NBUF = 6
_LAG = 3
_ACC_ROWS = _G + _NS
_ZROWS = _ACC_ROWS // _NS
_OROWS = _G // _NS

_mesh = plsc.VectorSubcoreMesh(core_axis_name="c", subcore_axis_name="s")


def _chunks_of(s):
    return _CPT if s < _NS - 1 else _NCH - _CPT * (_NS - 1)


_TAILG = _chunks_of(_NS - 1) - 1


@functools.partial(
    pl.kernel,
    out_type=jax.ShapeDtypeStruct((_G, _F), jnp.float32),
    mesh=_mesh,
    scratch_types=[
        pltpu.VMEM((_CPT, _CHUNK), jnp.int32),
        pltpu.VMEM((_NBUF, _CHUNK, _FH), jnp.float32),
        pltpu.VMEM_SHARED((_ACC_ROWS, _FH), jnp.float32),
    ]
    + [pltpu.SemaphoreType.DMA] * (2 * _NBUF),
    compiler_params=pltpu.CompilerParams(use_tc_tiling_on_sc=False),
)
def _segsum(x_hbm, bp_hbm, out_hbm, idx_v, rows_v, acc_sh, *sems):
    load_sems, add_sems = sems[:_NBUF], sems[_NBUF:]
    cid = lax.axis_index("c")
    sid = lax.axis_index("s")
    col0 = cid * _FH
    last = _NS - 1

    zero = jnp.zeros((_L,), jnp.float32)
    for i in range(_ZROWS):
        for j in range(_FH // _L):
            rows_v[0, i, pl.ds(j * _L, _L)] = zero
    pltpu.sync_copy(
        rows_v.at[0, pl.ds(0, _ZROWS)],
        acc_sh.at[pl.ds(sid * _ZROWS, _ZROWS)],
    )
    pltpu.sync_copy(bp_hbm.at[pl.ds(sid * _CPT, _CPT)], idx_v)

    def issue_load(g):
        b = g % _NBUF
        base = (sid * _CPT + g) * _CHUNK
        if g < _TAILG:
            return pltpu.async_copy(
                x_hbm.at[pl.ds(base, _CHUNK), pl.ds(col0, _FH)],
                rows_v.at[b],
                load_sems[b],
            )
        if g == _TAILG:
            @pl.when(sid < last)
            def _():
                pltpu.async_copy(
                    x_hbm.at[pl.ds(base, _CHUNK), pl.ds(col0, _FH)],
                    rows_v.at[b],
                    load_sems[b],
                )

            @pl.when(sid == last)
            def _():
                pltpu.async_copy(
                    x_hbm.at[pl.ds((_NCH - 1) * _CHUNK, _TAIL),
                             pl.ds(col0, _FH)],
                    rows_v.at[b, pl.ds(0, _TAIL)],
                    load_sems[b],
                )
            return None
        @pl.when(sid < last)
        def _():
            pltpu.async_copy(
                x_hbm.at[pl.ds(base, _CHUNK), pl.ds(col0, _FH)],
                rows_v.at[b],
                load_sems[b],
            )
        return None

    def wait_load(g):
        b = g % _NBUF
        if g < _TAILG:
            pltpu.make_async_copy(
                x_hbm.at[pl.ds(0, _CHUNK), pl.ds(0, _FH)],
                rows_v.at[b], load_sems[b]).wait()
        elif g == _TAILG:
            @pl.when(sid < last)
            def _():
                pltpu.make_async_copy(
                    x_hbm.at[pl.ds(0, _CHUNK), pl.ds(0, _FH)],
                    rows_v.at[b], load_sems[b]).wait()

            @pl.when(sid == last)
            def _():
                pltpu.make_async_copy(
                    x_hbm.at[pl.ds(0, _TAIL), pl.ds(0, _FH)],
                    rows_v.at[b, pl.ds(0, _TAIL)], load_sems[b]).wait()
        else:
            @pl.when(sid < last)
            def _():
                pltpu.make_async_copy(
                    x_hbm.at[pl.ds(0, _CHUNK), pl.ds(0, _FH)],
                    rows_v.at[b], load_sems[b]).wait()

    def issue_scatter(g):
        b = g % _NBUF
        def fire():
            pltpu.async_copy(
                rows_v.at[b], acc_sh.at[idx_v.at[g]], add_sems[b], add=True
            )
        if g <= _TAILG:
            fire()
        else:
            pl.when(sid < last)(fire)

    def wait_scatter(g):
        b = g % _NBUF
        def drain():
            pltpu.make_async_copy(
                x_hbm.at[pl.ds(0, _CHUNK), pl.ds(0, _FH)],
                rows_v.at[b], add_sems[b]).wait()
        if g <= _TAILG:
            drain()
        else:
            pl.when(sid < last)(drain)

    for g in range(_LAG):
        issue_load(g)
    plsc.subcore_barrier()

    for g in range(_CPT):
        wait_load(g)
        issue_scatter(g)
        if g >= _LAG:
            wait_scatter(g - _LAG)
        if g + _LAG < _CPT:
            issue_load(g + _LAG)
    for g in range(_CPT - _LAG, _CPT):
        wait_scatter(g)

    plsc.subcore_barrier()
    pltpu.sync_copy(
        acc_sh.at[pl.ds(sid * _OROWS, _OROWS)],
        out_hbm.at[pl.ds(sid * _OROWS, _OROWS), pl.ds(col0, _FH)],
    )


def kernel(x, edge_index, batch):
    pad = jnp.full((_NCH_PAD * _CHUNK - _NN,), _DUMMY, jnp.int32)
    batch_p = jnp.concatenate([batch, pad]).reshape(_NCH_PAD, _CHUNK)
    m = _segsum(x, batch_p)
    return (m, x)


# speedup vs baseline: 5.6570x; 1.4125x over previous
import functools

import jax
import jax.numpy as jnp
from jax import lax
from jax.experimental import pallas as pl
from jax.experimental.pallas import tpu as pltpu
from jax.experimental.pallas import tpu_sc as plsc

_NN = 100000
_F = 128
_G = 512
_NC = 2
_NS = 16
_L = 16
_FH = _F // _NC
_CHUNK = 128
_NCH = (_NN + _CHUNK - 1) // _CHUNK
_CPT = (_NCH + _NS - 1) // _NS
_NCH_PAD = _CPT * _NS
_TAIL = _NN - (_NCH - 1) * _CHUNK
_TAILG = _NCH - _CPT * (_NS - 1) - 1
_DUMMY = _G
_NBUF = 6
_LAG = 3
_ACC_ROWS = _G + _NS
_ZROWS = _ACC_ROWS // _NS
_OROWS = _G // _NS

_mesh = plsc.VectorSubcoreMesh(core_axis_name="c", subcore_axis_name="s")


@functools.partial(
    pl.kernel,
    out_type=(
        jax.ShapeDtypeStruct((_G, _F), jnp.float32),
        jax.ShapeDtypeStruct((_NN, _F), jnp.float32),
    ),
    mesh=_mesh,
    scratch_types=[
        pltpu.VMEM((_CPT, _CHUNK), jnp.int32),
        pltpu.VMEM((_NBUF, _CHUNK, _FH), jnp.float32),
        pltpu.VMEM_SHARED((_ACC_ROWS, _FH), jnp.float32),
    ]
    + [pltpu.SemaphoreType.DMA] * (3 * _NBUF),
    compiler_params=pltpu.CompilerParams(use_tc_tiling_on_sc=False),
)
def _segsum(x_hbm, bp_hbm, m_hbm, xo_hbm, idx_v, rows_v, acc_sh, *sems):
    load_sems = sems[:_NBUF]
    add_sems = sems[_NBUF:2 * _NBUF]
    wb_sems = sems[2 * _NBUF:]
    cid = lax.axis_index("c")
    sid = lax.axis_index("s")
    col0 = cid * _FH
    last = _NS - 1

    zero = jnp.zeros((_L,), jnp.float32)
    for i in range(_ZROWS):
        for j in range(_FH // _L):
            rows_v[0, i, pl.ds(j * _L, _L)] = zero
    pltpu.sync_copy(
        rows_v.at[0, pl.ds(0, _ZROWS)],
        acc_sh.at[pl.ds(sid * _ZROWS, _ZROWS)],
    )
    pltpu.sync_copy(bp_hbm.at[pl.ds(sid * _CPT, _CPT)], idx_v)

    def ranged(g, full, tail_variant):
        if g < _TAILG:
            full()
        elif g == _TAILG:
            pl.when(sid < last)(full)
            pl.when(sid == last)(tail_variant)
        else:
            pl.when(sid < last)(full)

    def issue_load(g):
        b = g % _NBUF
        base = (sid * _CPT + g) * _CHUNK

        def full():
            pltpu.async_copy(
                x_hbm.at[pl.ds(base, _CHUNK), pl.ds(col0, _FH)],
                rows_v.at[b],
                load_sems[b],
            )

        def tail():
            pltpu.async_copy(
                x_hbm.at[pl.ds((_NCH - 1) * _CHUNK, _TAIL), pl.ds(col0, _FH)],
                rows_v.at[b, pl.ds(0, _TAIL)],
                load_sems[b],
            )

        ranged(g, full, tail)

    def wait_dma(g, sem, rows_full, rows_tail):
        b = g % _NBUF

        def full():
            pltpu.make_async_copy(
                x_hbm.at[pl.ds(0, rows_full), pl.ds(0, _FH)],
                rows_v.at[b, pl.ds(0, rows_full)], sem).wait()

        def tail():
            pltpu.make_async_copy(
                x_hbm.at[pl.ds(0, rows_tail), pl.ds(0, _FH)],
                rows_v.at[b, pl.ds(0, rows_tail)], sem).wait()

        ranged(g, full, tail)

    def issue_scatter(g):
        b = g % _NBUF

        def fire():
            pltpu.async_copy(
                rows_v.at[b], acc_sh.at[idx_v.at[g]], add_sems[b], add=True
            )

        ranged(g, fire, fire)

    def issue_wb(g):
        b = g % _NBUF
        base = (sid * _CPT + g) * _CHUNK

        def full():
            pltpu.async_copy(
                rows_v.at[b],
                xo_hbm.at[pl.ds(base, _CHUNK), pl.ds(col0, _FH)],
                wb_sems[b],
            )

        def tail():
            pltpu.async_copy(
                rows_v.at[b, pl.ds(0, _TAIL)],
                xo_hbm.at[pl.ds((_NCH - 1) * _CHUNK, _TAIL),
                          pl.ds(col0, _FH)],
                wb_sems[b],
            )

        ranged(g, full, tail)

    for g in range(_LAG):
        issue_load(g)
    plsc.subcore_barrier()

    for g in range(_CPT):
        wait_dma(g, load_sems[g % _NBUF], _CHUNK, _TAIL)
        issue_scatter(g)
        issue_wb(g)
        if g >= _LAG:
            gp = g - _LAG
            wait_dma(gp, add_sems[gp % _NBUF], _CHUNK, _CHUNK)
            wait_dma(gp, wb_sems[gp % _NBUF], _CHUNK, _TAIL)
        if g + _LAG < _CPT:
            issue_load(g + _LAG)
    for g in range(_CPT - _LAG, _CPT):
        wait_dma(g, add_sems[g % _NBUF], _CHUNK, _CHUNK)
        wait_dma(g, wb_sems[g % _NBUF], _CHUNK, _TAIL)

    plsc.subcore_barrier()
    pltpu.sync_copy(
        acc_sh.at[pl.ds(sid * _OROWS, _OROWS)],
        m_hbm.at[pl.ds(sid * _OROWS, _OROWS), pl.ds(col0, _FH)],
    )


def kernel(x, edge_index, batch):
    pad = jnp.full((_NCH_PAD * _CHUNK - _NN,), _DUMMY, jnp.int32)
    batch_p = jnp.concatenate([batch, pad]).reshape(_NCH_PAD, _CHUNK)
    m, x_out = _segsum(x, batch_p)
    return (m, x_out)
